# Initial kernel scaffold; baseline (speedup 1.0000x reference)
import jax
import jax.numpy as jnp
from jax.experimental import pallas as pl


def kernel(x, token_embedding):
    raise NotImplementedError("write your pallas kernel here")


# speedup vs baseline: 1.2363x; 1.2363x over previous
import functools
import jax
import jax.numpy as jnp
from jax import lax
from jax.experimental import pallas as pl
from jax.experimental.pallas import tpu as pltpu
from jax.experimental.pallas import tpu_sc as plsc

EMBED_DIM = 768
B_TOTAL = 1024 * 77
NUM_WORKERS = 32
B_PER_W = B_TOTAL // NUM_WORKERS
CHUNK = 56
NCHUNK = B_PER_W // CHUNK


def _sc_gather(table, idx):
  mesh = plsc.VectorSubcoreMesh(core_axis_name="c", subcore_axis_name="s")

  @functools.partial(
      pl.kernel,
      mesh=mesh,
      out_type=jax.ShapeDtypeStruct((B_TOTAL, EMBED_DIM), jnp.float32),
      scratch_types=[
          pltpu.VMEM((B_PER_W,), jnp.int32),
          pltpu.VMEM((CHUNK, EMBED_DIM), jnp.float32),
          pltpu.SemaphoreType.DMA,
      ],
  )
  def k(table_hbm, idx_hbm, out_hbm, idx_v, rows_v, sem):
    wid = lax.axis_index("s") * 2 + lax.axis_index("c")
    base = wid * B_PER_W
    pltpu.sync_copy(idx_hbm.at[pl.ds(base, B_PER_W)], idx_v)

    def chunk_body(j, carry):
      off = pl.multiple_of(j * CHUNK, 8)
      pltpu.async_copy(
          table_hbm.at[idx_v.at[pl.ds(off, CHUNK)]], rows_v, sem).wait()
      pltpu.sync_copy(rows_v, out_hbm.at[pl.ds(base + off, CHUNK)])
      return carry

    lax.fori_loop(0, NCHUNK, chunk_body, 0)

  return k(table, idx)


def kernel(x, token_embedding):
  idx = x.reshape(-1).astype(jnp.int32)
  out = _sc_gather(token_embedding, idx)
  return out.reshape(x.shape[0], x.shape[1], EMBED_DIM)


# speedup vs baseline: 1.2977x; 1.0496x over previous
import functools
import jax
import jax.numpy as jnp
from jax import lax
from jax.experimental import pallas as pl
from jax.experimental.pallas import tpu as pltpu
from jax.experimental.pallas import tpu_sc as plsc

EMBED_DIM = 768
B_TOTAL = 1024 * 77
NUM_WORKERS = 32
B_PER_W = B_TOTAL // NUM_WORKERS
CHUNK = 56
NCHUNK = B_PER_W // CHUNK


def _sc_gather(table, idx):
  mesh = plsc.VectorSubcoreMesh(core_axis_name="c", subcore_axis_name="s")

  @functools.partial(
      pl.kernel,
      mesh=mesh,
      out_type=jax.ShapeDtypeStruct((B_TOTAL, EMBED_DIM), jnp.float32),
      scratch_types=[
          pltpu.VMEM((B_PER_W,), jnp.int32),
          pltpu.VMEM((CHUNK, EMBED_DIM), jnp.float32),
          pltpu.VMEM((CHUNK, EMBED_DIM), jnp.float32),
          pltpu.SemaphoreType.DMA,
          pltpu.SemaphoreType.DMA,
          pltpu.SemaphoreType.DMA,
          pltpu.SemaphoreType.DMA,
      ],
  )
  def k(table_hbm, idx_hbm, out_hbm, idx_v, buf0, buf1,
        gsem0, gsem1, ssem0, ssem1):
    wid = lax.axis_index("s") * 2 + lax.axis_index("c")
    base = wid * B_PER_W
    pltpu.sync_copy(idx_hbm.at[pl.ds(base, B_PER_W)], idx_v)

    bufs = (buf0, buf1)
    gsems = (gsem0, gsem1)
    ssems = (ssem0, ssem1)

    def gather(j, p):
      off = pl.multiple_of(j * CHUNK, 8)
      return pltpu.async_copy(
          table_hbm.at[idx_v.at[pl.ds(off, CHUNK)]], bufs[p], gsems[p])

    def store(j, p):
      off = pl.multiple_of(j * CHUNK, 8)
      return pltpu.async_copy(
          bufs[p], out_hbm.at[pl.ds(base + off, CHUNK)], ssems[p])

    gather(0, 0).wait()
    gather(1, 1)
    store(0, 0)

    def pair_body(m, carry):
      g = 1 + 2 * m
      for (j, p) in ((g, 1), (g + 1, 0)):
        q = 1 - p
        pltpu.make_async_copy(
            table_hbm.at[idx_v.at[pl.ds(pl.multiple_of(j * CHUNK, 8), CHUNK)]],
            bufs[p], gsems[p]).wait()
        pltpu.make_async_copy(
            bufs[q],
            out_hbm.at[pl.ds(base + pl.multiple_of((j - 1) * CHUNK, 8), CHUNK)],
            ssems[q]).wait()
        gather(j + 1, q)
        store(j, p)
      return carry

    lax.fori_loop(0, (NCHUNK - 2) // 2, pair_body, 0, unroll=False)

    j_last = NCHUNK - 1
    pltpu.make_async_copy(
        table_hbm.at[idx_v.at[pl.ds(pl.multiple_of(j_last * CHUNK, 8), CHUNK)]],
        bufs[1], gsems[1]).wait()
    pltpu.make_async_copy(
        bufs[0],
        out_hbm.at[pl.ds(base + pl.multiple_of((j_last - 1) * CHUNK, 8), CHUNK)],
        ssems[0]).wait()
    store(j_last, 1)
    pltpu.make_async_copy(
        bufs[1],
        out_hbm.at[pl.ds(base + pl.multiple_of(j_last * CHUNK, 8), CHUNK)],
        ssems[1]).wait()

  return k(table, idx)


def kernel(x, token_embedding):
  idx = x.reshape(-1).astype(jnp.int32)
  out = _sc_gather(token_embedding, idx)
  return out.reshape(x.shape[0], x.shape[1], EMBED_DIM)
